# baseline (speedup 1.0000x reference)
import dataclasses
import functools

import jax
import jax.numpy as jnp
from jax.experimental import pallas as pl
from jax.experimental.pallas import tpu as pltpu
from jax.experimental.pallas import tpu_sc as plsc

N = 10000
E = 320000
H = 128
B = 16
NP = 10240
NC = 2
NS = 16
NW = NC * NS
EW = E // NW
GC = 128
NCH = EW // GC
GT = EW - NCH * GC
PT = 4
KS = 1
SW = H // KS
CW = 16
ECS = E // NC
GS = 64
NCH2 = ECS // GS
BE = 1600
RN = 1280
NBE = E // BE


def _silu(x):
    return x * jax.lax.logistic(x)


@functools.lru_cache(maxsize=1)
def _sc_kernels():
    mesh = plsc.VectorSubcoreMesh(core_axis_name="c", subcore_axis_name="s")
    cp = pltpu.CompilerParams()
    if "needs_layout_passes" in pltpu.CompilerParams.__dataclass_fields__:
        cp = dataclasses.replace(cp, needs_layout_passes=False)
    cp2 = dataclasses.replace(cp, use_tc_tiling_on_sc=False)

    @functools.partial(
        pl.kernel,
        compiler_params=cp,
        out_type=(
            jax.ShapeDtypeStruct((E, H), jnp.float32),
            jax.ShapeDtypeStruct((E, H), jnp.float32),
            jax.ShapeDtypeStruct((E,), jnp.float32),
            jax.ShapeDtypeStruct((E,), jnp.float32),
            jax.ShapeDtypeStruct((E,), jnp.float32),
        ),
        mesh=mesh,
        scratch_types=[
            pltpu.VMEM((2, GC), jnp.int32),
            pltpu.VMEM((2, GC), jnp.int32),
            pltpu.VMEM((2, GC, H), jnp.float32),
            pltpu.VMEM((2, GC, H), jnp.float32),
            pltpu.VMEM((2, GC), jnp.float32),
            pltpu.VMEM((2, GC), jnp.float32),
            pltpu.VMEM((2, GC), jnp.float32),
            pltpu.VMEM((NP * PT,), jnp.float32),
            pltpu.SemaphoreType.DMA,
            pltpu.SemaphoreType.DMA,
            pltpu.SemaphoreType.DMA,
            pltpu.SemaphoreType.DMA,
            pltpu.SemaphoreType.DMA,
            pltpu.SemaphoreType.DMA,
        ],
    )
    def sc_gather(a_hbm, b_hbm, p_hbm, dst_hbm, src_hbm,
                  ad_hbm, bs_hbm, rx_hbm, ry_hbm, rz_hbm,
                  idx_d, idx_s, buf_a, buf_b, buf_x, buf_y, buf_z,
                  ptab, gsem0, gsem1, isem0, isem1, wsem0, wsem1):
        cid = jax.lax.axis_index("c")
        sid = jax.lax.axis_index("s")
        base = (cid * NS + sid) * EW
        pltpu.sync_copy(p_hbm, ptab)
        isem = (isem0, isem1)
        wsem = (wsem0, wsem1)
        gsem = (gsem0, gsem1)

        def fire_idx(ci, b):
            off = base + ci * GC
            pltpu.async_copy(dst_hbm.at[pl.ds(off, GC)], idx_d.at[b], isem[b])
            pltpu.async_copy(src_hbm.at[pl.ds(off, GC)], idx_s.at[b], isem[b])

        def drain_idx(ci, b):
            off = base + ci * GC
            pltpu.make_async_copy(dst_hbm.at[pl.ds(off, GC)], idx_d.at[b],
                                  isem[b]).wait()
            pltpu.make_async_copy(src_hbm.at[pl.ds(off, GC)], idx_s.at[b],
                                  isem[b]).wait()

        def fire_writes(ci, b):
            off = base + ci * GC
            pltpu.async_copy(buf_a.at[b], ad_hbm.at[pl.ds(off, GC)], wsem[b])
            pltpu.async_copy(buf_b.at[b], bs_hbm.at[pl.ds(off, GC)], wsem[b])
            pltpu.async_copy(buf_x.at[b], rx_hbm.at[pl.ds(off, GC)], wsem[b])
            pltpu.async_copy(buf_y.at[b], ry_hbm.at[pl.ds(off, GC)], wsem[b])
            pltpu.async_copy(buf_z.at[b], rz_hbm.at[pl.ds(off, GC)], wsem[b])

        def drain_writes(ci, b):
            off = base + ci * GC
            pltpu.make_async_copy(buf_a.at[b], ad_hbm.at[pl.ds(off, GC)],
                                  wsem[b]).wait()
            pltpu.make_async_copy(buf_b.at[b], bs_hbm.at[pl.ds(off, GC)],
                                  wsem[b]).wait()
            pltpu.make_async_copy(buf_x.at[b], rx_hbm.at[pl.ds(off, GC)],
                                  wsem[b]).wait()
            pltpu.make_async_copy(buf_y.at[b], ry_hbm.at[pl.ds(off, GC)],
                                  wsem[b]).wait()
            pltpu.make_async_copy(buf_z.at[b], rz_hbm.at[pl.ds(off, GC)],
                                  wsem[b]).wait()

        def fire_gathers(b):
            pltpu.async_copy(a_hbm.at[idx_d.at[b]], buf_a.at[b], gsem[b])
            pltpu.async_copy(b_hbm.at[idx_s.at[b]], buf_b.at[b], gsem[b])

        def drain_gathers(b):
            pltpu.make_async_copy(a_hbm.at[idx_d.at[b]], buf_a.at[b],
                                  gsem[b]).wait()
            pltpu.make_async_copy(b_hbm.at[idx_s.at[b]], buf_b.at[b],
                                  gsem[b]).wait()

        def body(ci, b):
            @pl.when(ci > 0)
            def _():
                drain_writes(ci - 1, 1 - b)

            @pl.when(ci + 1 < NCH)
            def _():
                drain_idx(ci + 1, 1 - b)
                fire_gathers(1 - b)

            drain_gathers(b)

            @pl.when(ci + 2 < NCH)
            def _():
                fire_idx(ci + 2, b)

            for k in range(GC // 16):
                sl = pl.ds(k * 16, 16)
                ird = idx_d[b, sl] * PT
                irs = idx_s[b, sl] * PT
                buf_x[b, sl] = (plsc.load_gather(ptab, [ird])
                                - plsc.load_gather(ptab, [irs]))
                buf_y[b, sl] = (plsc.load_gather(ptab, [ird + 1])
                                - plsc.load_gather(ptab, [irs + 1]))
                buf_z[b, sl] = (plsc.load_gather(ptab, [ird + 2])
                                - plsc.load_gather(ptab, [irs + 2]))

            fire_writes(ci, b)

        fire_idx(0, 0)
        fire_idx(1, 1)
        drain_idx(0, 0)
        fire_gathers(0)

        @pl.loop(0, NCH // 2)
        def _(k):
            body(2 * k, 0)
            body(2 * k + 1, 1)

        drain_writes(NCH - 1, 1)

        toff = base + NCH * GC
        tsl = pl.ds(0, GT)
        pltpu.sync_copy(dst_hbm.at[pl.ds(toff, GT)], idx_d.at[0, tsl])
        pltpu.sync_copy(src_hbm.at[pl.ds(toff, GT)], idx_s.at[0, tsl])
        pltpu.sync_copy(a_hbm.at[idx_d.at[0, tsl]], buf_a.at[0, tsl])
        pltpu.sync_copy(b_hbm.at[idx_s.at[0, tsl]], buf_b.at[0, tsl])
        ird = idx_d[0, tsl] * PT
        irs = idx_s[0, tsl] * PT
        buf_x[0, tsl] = (plsc.load_gather(ptab, [ird])
                         - plsc.load_gather(ptab, [irs]))
        buf_y[0, tsl] = (plsc.load_gather(ptab, [ird + 1])
                         - plsc.load_gather(ptab, [irs + 1]))
        buf_z[0, tsl] = (plsc.load_gather(ptab, [ird + 2])
                         - plsc.load_gather(ptab, [irs + 2]))
        pltpu.sync_copy(buf_a.at[0, tsl], ad_hbm.at[pl.ds(toff, GT)])
        pltpu.sync_copy(buf_b.at[0, tsl], bs_hbm.at[pl.ds(toff, GT)])
        pltpu.sync_copy(buf_x.at[0, tsl], rx_hbm.at[pl.ds(toff, GT)])
        pltpu.sync_copy(buf_y.at[0, tsl], ry_hbm.at[pl.ds(toff, GT)])
        pltpu.sync_copy(buf_z.at[0, tsl], rz_hbm.at[pl.ds(toff, GT)])

    mesh2 = plsc.VectorSubcoreMesh(core_axis_name="c", subcore_axis_name="s",
                                   num_cores=NC, num_subcores=KS)

    @functools.partial(
        pl.kernel,
        compiler_params=cp2,
        out_type=(
            jax.ShapeDtypeStruct((NC, NP, H), jnp.float32),
            jax.ShapeDtypeStruct((NC * KS, NP, H), jnp.float32),
        ),
        mesh=mesh2,
        scratch_types=[
            pltpu.VMEM((3, GS), jnp.int32),
            pltpu.VMEM((3, GS, SW), jnp.float32),
            pltpu.VMEM((3, GS, CW), jnp.float32),
            pltpu.VMEM((32, SW), jnp.float32),
            pltpu.VMEM_SHARED((NP, SW), jnp.float32),
            pltpu.VMEM_SHARED((NP, CW), jnp.float32),
            pltpu.SemaphoreType.DMA,
            pltpu.SemaphoreType.DMA,
            pltpu.SemaphoreType.DMA,
            pltpu.SemaphoreType.DMA,
            pltpu.SemaphoreType.DMA,
            pltpu.SemaphoreType.DMA,
            pltpu.SemaphoreType.DMA,
        ],
    )
    def sc_scatter(ef_hbm, cd_hbm, dst_hbm,
                   hpart_hbm, ppart_hbm,
                   idx_v, buf_ef, buf_cd, zbuf, acc_h, acc_p,
                   zsem, ssem0, ssem1, ssem2, lsem0, lsem1, lsem2):
        cid = jax.lax.axis_index("c")
        sid = jax.lax.axis_index("s")
        lsem = (lsem0, lsem1, lsem2)
        ssem = (ssem0, ssem1, ssem2)

        z16 = jnp.zeros((16,), jnp.float32)

        @pl.loop(0, 32)
        def _(j):
            @pl.loop(0, SW // 16)
            def _(q):
                zbuf[j, pl.ds(q * 16, 16)] = z16

        @pl.loop(0, NP // 32)
        def _(k):
            pltpu.async_copy(zbuf, acc_h.at[pl.ds(k * 32, 32)], zsem)
            pltpu.async_copy(zbuf.at[:, pl.ds(0, CW)],
                             acc_p.at[pl.ds(k * 32, 32)], zsem)

        @pl.loop(0, NP // 32)
        def _(k):
            pltpu.make_async_copy(zbuf, acc_h.at[pl.ds(k * 32, 32)],
                                  zsem).wait()
            pltpu.make_async_copy(zbuf.at[:, pl.ds(0, CW)],
                                  acc_p.at[pl.ds(k * 32, 32)], zsem).wait()

        ebase = cid * ECS
        lane0 = sid * SW

        def fire_loads(ci, b):
            off = ebase + ci * GS
            pltpu.async_copy(dst_hbm.at[pl.ds(off, GS)], idx_v.at[b], lsem[b])
            pltpu.async_copy(ef_hbm.at[pl.ds(off, GS), pl.ds(lane0, SW)],
                             buf_ef.at[b], lsem[b])
            pltpu.async_copy(cd_hbm.at[pl.ds(off, GS), pl.ds(0, CW)],
                             buf_cd.at[b], lsem[b])

        def drain_loads(ci, b):
            off = ebase + ci * GS
            pltpu.make_async_copy(dst_hbm.at[pl.ds(off, GS)], idx_v.at[b],
                                  lsem[b]).wait()
            pltpu.make_async_copy(ef_hbm.at[pl.ds(off, GS),
                                            pl.ds(lane0, SW)],
                                  buf_ef.at[b], lsem[b]).wait()
            pltpu.make_async_copy(cd_hbm.at[pl.ds(off, GS), pl.ds(0, CW)],
                                  buf_cd.at[b], lsem[b]).wait()

        def drain_streams(b):
            pltpu.make_async_copy(buf_ef.at[b], acc_h.at[idx_v.at[b]],
                                  ssem[b]).wait()
            pltpu.make_async_copy(buf_cd.at[b], acc_p.at[idx_v.at[b]],
                                  ssem[b]).wait()

        def body(ci, b):
            drain_loads(ci, b)
            pltpu.async_copy(buf_ef.at[b], acc_h.at[idx_v.at[b]],
                             ssem[b], add=True)
            pltpu.async_copy(buf_cd.at[b], acc_p.at[idx_v.at[b]],
                             ssem[b], add=True)
            pb = (b + 2) % 3

            @pl.when(ci > 0)
            def _():
                drain_streams(pb)

            @pl.when(ci + 2 < NCH2)
            def _():
                fire_loads(ci + 2, pb)

        fire_loads(0, 0)
        fire_loads(1, 1)

        @pl.loop(0, NCH2 // 3)
        def _(k):
            body(3 * k, 0)
            body(3 * k + 1, 1)
            body(3 * k + 2, 2)

        body(NCH2 - 1, 0)
        drain_streams(0)

        @pl.loop(0, NP // 640)
        def _(k):
            r = k * 640
            pltpu.sync_copy(acc_h.at[pl.ds(r, 640)],
                            hpart_hbm.at[cid, pl.ds(r, 640),
                                         pl.ds(lane0, SW)])
            pltpu.sync_copy(acc_p.at[pl.ds(r, 640)],
                            ppart_hbm.at[cid * KS + sid, pl.ds(r, 640),
                                         pl.ds(0, CW)])

    return sc_gather, sc_scatter


def _edge_body(ad, bs, rx, ry, rz, w1c, ew2, eb2, cw1, cb1, cw2r, ef_o, cd_o):
    vx = rx[0]
    vy = ry[0]
    vz = rz[0]
    dist = jnp.sqrt(vx * vx + vy * vy + vz * vz + 1e-8)
    dn = (((0,), (0,)), ((), ()))
    dterm = jax.lax.dot_general(dist, w1c[...], dn,
                                preferred_element_type=jnp.float32)
    x1 = _silu(ad[...] + bs[...] + dterm)
    ef = _silu(jnp.dot(x1, ew2[...], preferred_element_type=jnp.float32) + eb2[...])
    t = _silu(jnp.dot(ef, cw1[...], preferred_element_type=jnp.float32) + cb1[...])
    cw = jnp.sum(t * cw2r[...], axis=1, keepdims=True)
    rows = jnp.concatenate(
        [vx, vy, vz, jnp.zeros((5, vx.shape[1]), jnp.float32)], axis=0)
    ipad = jnp.eye(8, H, dtype=jnp.float32)
    rel_cols = jax.lax.dot_general(rows, ipad, dn,
                                   preferred_element_type=jnp.float32)
    ef_o[...] = ef
    cd_o[...] = cw * rel_cols


def _edge_call(ad, bs, rx3, ry3, rz3, w1c, ew2, eb2, cw1, cb1, cw2r):
    row = lambda i: (i, 0)
    full = lambda i: (0, 0)
    vec = lambda i: (i, 0, 0)
    return pl.pallas_call(
        _edge_body,
        grid=(NBE,),
        in_specs=[
            pl.BlockSpec((BE, H), row),
            pl.BlockSpec((BE, H), row),
            pl.BlockSpec((1, 1, BE), vec),
            pl.BlockSpec((1, 1, BE), vec),
            pl.BlockSpec((1, 1, BE), vec),
            pl.BlockSpec((1, H), full),
            pl.BlockSpec((H, H), full),
            pl.BlockSpec((1, H), full),
            pl.BlockSpec((H, H), full),
            pl.BlockSpec((1, H), full),
            pl.BlockSpec((1, H), full),
        ],
        out_specs=[
            pl.BlockSpec((BE, H), row),
            pl.BlockSpec((BE, H), row),
        ],
        out_shape=[
            jax.ShapeDtypeStruct((E, H), jnp.float32),
            jax.ShapeDtypeStruct((E, H), jnp.float32),
        ],
    )(ad, bs, rx3, ry3, rz3, w1c, ew2, eb2, cw1, cb1, cw2r)


def _precompute(hn, w1a, w1b, w1c, eb1, ew2, eb2):
    a = jnp.dot(hn, w1a[...], preferred_element_type=jnp.float32) + eb1[...]
    bm = jnp.dot(hn, w1b[...], preferred_element_type=jnp.float32)
    dist0 = jnp.sqrt(jnp.float32(1e-8))
    efl = _silu(jnp.dot(_silu(a + bm + dist0 * w1c[...]), ew2[...],
                        preferred_element_type=jnp.float32) + eb2[...])
    return a, bm, efl


def _embed_body(nf, inw, inb, w1a, w1b, w1c, eb1, ew2, eb2,
                h_o, a_o, b_o, efl_o):
    h = jnp.dot(nf[...], inw[...], preferred_element_type=jnp.float32) + inb[...]
    a, bm, efl = _precompute(h, w1a, w1b, w1c, eb1, ew2, eb2)
    h_o[...] = h
    a_o[...] = a
    b_o[...] = bm
    efl_o[...] = efl


def _embed_call(nf_p, inw, inb, w1a, w1b, w1c, eb1, ew2, eb2):
    row = lambda i: (i, 0)
    full = lambda i: (0, 0)
    return pl.pallas_call(
        _embed_body,
        grid=(NP // RN,),
        in_specs=[
            pl.BlockSpec((RN, H), row),
            pl.BlockSpec((H, H), full),
            pl.BlockSpec((1, H), full),
            pl.BlockSpec((H, H), full),
            pl.BlockSpec((H, H), full),
            pl.BlockSpec((1, H), full),
            pl.BlockSpec((1, H), full),
            pl.BlockSpec((H, H), full),
            pl.BlockSpec((1, H), full),
        ],
        out_specs=[pl.BlockSpec((RN, H), row)] * 4,
        out_shape=[jax.ShapeDtypeStruct((NP, H), jnp.float32)] * 4,
    )(nf_p, inw, inb, w1a, w1b, w1c, eb1, ew2, eb2)


def _node_update(h, hag, nw1a, nw1b, nb1, nw2, nb2, lng, lnb):
    u = _silu(jnp.dot(h, nw1a[...], preferred_element_type=jnp.float32)
              + jnp.dot(hag, nw1b[...], preferred_element_type=jnp.float32)
              + nb1[...])
    u = jnp.dot(u, nw2[...], preferred_element_type=jnp.float32) + nb2[...]
    x = h + h + u
    mu = jnp.mean(x, axis=1, keepdims=True)
    xc = x - mu
    var = jnp.mean(xc * xc, axis=1, keepdims=True)
    return xc * jax.lax.rsqrt(var + 1e-5) * lng[...] + lnb[...]


def _node_mid_body(h, hp0, hp1, efl, p, pp,
                   nw1a, nw1b, nb1, nw2, nb2, lng, lnb,
                   w1a, w1b, w1c, eb1, ew2, eb2,
                   h_o, p_o, a_o, b_o, efl_o):
    hag = hp0[0] + hp1[0] + efl[...]
    hn = _node_update(h[...], hag, nw1a, nw1b, nb1, nw2, nb2, lng, lnb)
    a, bm, efl2 = _precompute(hn, w1a, w1b, w1c, eb1, ew2, eb2)
    h_o[...] = hn
    p_o[...] = p[...] + jnp.sum(pp[...], axis=0)
    a_o[...] = a
    b_o[...] = bm
    efl_o[...] = efl2


def _node_mid_call(h, hparts, efl, p, pparts, nw1a, nw1b, nb1, nw2, nb2,
                   lng, lnb, w1a, w1b, w1c, eb1, ew2, eb2):
    row = lambda i: (i, 0)
    full = lambda i: (0, 0)
    part = lambda c: (lambda i: (c, i, 0))
    return pl.pallas_call(
        _node_mid_body,
        grid=(NP // RN,),
        in_specs=[
            pl.BlockSpec((RN, H), row),
            pl.BlockSpec((1, RN, H), part(0)),
            pl.BlockSpec((1, RN, H), part(1)),
            pl.BlockSpec((RN, H), row),
            pl.BlockSpec((RN, PT), row),
            pl.BlockSpec((NC * KS, RN, PT), lambda i: (0, i, 0)),
            pl.BlockSpec((H, H), full),
            pl.BlockSpec((H, H), full),
            pl.BlockSpec((1, H), full),
            pl.BlockSpec((H, H), full),
            pl.BlockSpec((1, H), full),
            pl.BlockSpec((1, H), full),
            pl.BlockSpec((1, H), full),
            pl.BlockSpec((H, H), full),
            pl.BlockSpec((H, H), full),
            pl.BlockSpec((1, H), full),
            pl.BlockSpec((1, H), full),
            pl.BlockSpec((H, H), full),
            pl.BlockSpec((1, H), full),
        ],
        out_specs=[
            pl.BlockSpec((RN, H), row),
            pl.BlockSpec((RN, PT), row),
            pl.BlockSpec((RN, H), row),
            pl.BlockSpec((RN, H), row),
            pl.BlockSpec((RN, H), row),
        ],
        out_shape=[
            jax.ShapeDtypeStruct((NP, H), jnp.float32),
            jax.ShapeDtypeStruct((NP, PT), jnp.float32),
            jax.ShapeDtypeStruct((NP, H), jnp.float32),
            jax.ShapeDtypeStruct((NP, H), jnp.float32),
            jax.ShapeDtypeStruct((NP, H), jnp.float32),
        ],
    )(h, hparts, hparts, efl, p, pparts,
      nw1a, nw1b, nb1, nw2, nb2, lng, lnb, w1a, w1b, w1c, eb1, ew2, eb2)


def _node_final_body(h, hp0, hp1, efl,
                     nw1a, nw1b, nb1, nw2, nb2, lng, lnb, h_o):
    hag = hp0[0] + hp1[0] + efl[...]
    h_o[...] = _node_update(h[...], hag, nw1a, nw1b, nb1, nw2, nb2, lng, lnb)


def _node_final_call(h, hparts, efl, nw1a, nw1b, nb1, nw2, nb2, lng, lnb):
    row = lambda i: (i, 0)
    full = lambda i: (0, 0)
    part = lambda c: (lambda i: (c, i, 0))
    return pl.pallas_call(
        _node_final_body,
        grid=(NP // RN,),
        in_specs=[
            pl.BlockSpec((RN, H), row),
            pl.BlockSpec((1, RN, H), part(0)),
            pl.BlockSpec((1, RN, H), part(1)),
            pl.BlockSpec((RN, H), row),
            pl.BlockSpec((H, H), full),
            pl.BlockSpec((H, H), full),
            pl.BlockSpec((1, H), full),
            pl.BlockSpec((H, H), full),
            pl.BlockSpec((1, H), full),
            pl.BlockSpec((1, H), full),
            pl.BlockSpec((1, H), full),
        ],
        out_specs=[pl.BlockSpec((RN, H), row)],
        out_shape=[jax.ShapeDtypeStruct((NP, H), jnp.float32)],
    )(h, hparts, hparts, efl, nw1a, nw1b, nb1, nw2, nb2, lng, lnb)[0]


def _pool_body(h, bat, outw, outb, o_ref, acc, cnt):
    i = pl.program_id(0)

    @pl.when(i == 0)
    def _():
        acc[...] = jnp.zeros_like(acc)
        cnt[...] = jnp.zeros_like(cnt)

    bb = bat[0]
    oh = (jax.lax.broadcasted_iota(jnp.int32, (B, RN), 0) == bb).astype(jnp.float32)
    acc[...] += jnp.dot(oh, h[...], preferred_element_type=jnp.float32)
    cnt[...] += jnp.sum(oh, axis=1, keepdims=True)

    @pl.when(i == pl.num_programs(0) - 1)
    def _():
        hg = acc[...] / cnt[...]
        o_ref[...] = jnp.dot(hg, outw[...],
                             preferred_element_type=jnp.float32) + outb[...]


def _pool_call(h, bat3, outw, outb):
    row = lambda i: (i, 0)
    full = lambda i: (0, 0)
    return pl.pallas_call(
        _pool_body,
        grid=(NP // RN,),
        in_specs=[
            pl.BlockSpec((RN, H), row),
            pl.BlockSpec((1, 1, RN), lambda i: (i, 0, 0)),
            pl.BlockSpec((H, H), full),
            pl.BlockSpec((1, H), full),
        ],
        out_specs=pl.BlockSpec((B, H), full),
        out_shape=jax.ShapeDtypeStruct((B, H), jnp.float32),
        scratch_shapes=[
            pltpu.VMEM((B, H), jnp.float32),
            pltpu.VMEM((B, 1), jnp.float32),
        ],
    )(h, bat3, outw, outb)


def kernel(node_features, pos, edge_index, batch, params):
    f32 = jnp.float32
    nf_p = jnp.pad(node_features, ((0, NP - N), (0, 0)))
    p4 = jnp.pad(pos.astype(f32), ((0, NP - N), (0, PT - 3)))
    bat3 = jnp.pad(batch.astype(jnp.int32), (0, NP - N),
                   constant_values=B).reshape(NP // RN, 1, RN)
    src = edge_index[0]
    dst = edge_index[1]

    def esplit(lp):
        w1 = lp['e_w1']
        return (w1[:H], w1[H:2 * H], w1[2 * H:2 * H + 1],
                lp['e_b1'].reshape(1, H), lp['e_w2'], lp['e_b2'].reshape(1, H))

    l0 = params['layers'][0]
    w1a, w1b, w1c, eb1, ew2, eb2 = esplit(l0)
    h, a, bm, efl = _embed_call(
        nf_p, params['in_w'], params['in_b'].reshape(1, H),
        w1a, w1b, w1c, eb1, ew2, eb2)
    p = p4

    for li in range(len(params['layers'])):
        lp = params['layers'][li]
        w1a, w1b, w1c, eb1, ew2, eb2 = esplit(lp)
        sc_gather, sc_scatter = _sc_kernels()
        ad, bs, rx, ry, rz = sc_gather(a, bm, p.reshape(NP * PT), dst, src)
        rx3 = rx.reshape(NBE, 1, BE)
        ry3 = ry.reshape(NBE, 1, BE)
        rz3 = rz.reshape(NBE, 1, BE)
        ef, cd = _edge_call(ad, bs, rx3, ry3, rz3, w1c, ew2, eb2,
                            lp['c_w1'], lp['c_b1'].reshape(1, H),
                            lp['c_w2'].reshape(1, H))
        hparts, pparts = sc_scatter(ef, cd, dst)
        pp4 = pparts[:, :, :PT]
        nw1 = lp['n_w1']
        args = (nw1[:H], nw1[H:2 * H], lp['n_b1'].reshape(1, H),
                lp['n_w2'], lp['n_b2'].reshape(1, H),
                lp['ln_g'].reshape(1, H), lp['ln_b'].reshape(1, H))
        if li + 1 < len(params['layers']):
            nxt = esplit(params['layers'][li + 1])
            h, p, a, bm, efl = _node_mid_call(h, hparts, efl, p, pp4,
                                             *args, *nxt)
        else:
            h = _node_final_call(h, hparts, efl, *args)

    return _pool_call(h, bat3, params['out_w'],
                      params['out_b'].reshape(1, H))


# speedup vs baseline: 1.0024x; 1.0024x over previous
import dataclasses
import functools

import jax
import jax.numpy as jnp
from jax.experimental import pallas as pl
from jax.experimental.pallas import tpu as pltpu
from jax.experimental.pallas import tpu_sc as plsc

N = 10000
E = 320000
H = 128
B = 16
NP = 10240
NC = 2
NS = 16
NW = NC * NS
EW = E // NW
GC = 80
NCH = EW // GC
PT = 4
KS = 1
SW = H // KS
CW = 16
ECS = E // NC
GS = 64
NCH2 = ECS // GS
BE = 1600
RN = 1280
NBE = E // BE


def _silu(x):
    return x * jax.lax.logistic(x)


@functools.lru_cache(maxsize=1)
def _sc_kernels():
    mesh = plsc.VectorSubcoreMesh(core_axis_name="c", subcore_axis_name="s")
    cp = pltpu.CompilerParams()
    if "needs_layout_passes" in pltpu.CompilerParams.__dataclass_fields__:
        cp = dataclasses.replace(cp, needs_layout_passes=False)
    cp2 = dataclasses.replace(cp, use_tc_tiling_on_sc=False)

    @functools.partial(
        pl.kernel,
        compiler_params=cp,
        out_type=(
            jax.ShapeDtypeStruct((E, H), jnp.float32),
            jax.ShapeDtypeStruct((E, H), jnp.float32),
            jax.ShapeDtypeStruct((E,), jnp.float32),
            jax.ShapeDtypeStruct((E,), jnp.float32),
            jax.ShapeDtypeStruct((E,), jnp.float32),
        ),
        mesh=mesh,
        scratch_types=[
            pltpu.VMEM((3, GC), jnp.int32),
            pltpu.VMEM((3, GC), jnp.int32),
            pltpu.VMEM((3, GC, H), jnp.float32),
            pltpu.VMEM((3, GC, H), jnp.float32),
            pltpu.VMEM((3, GC), jnp.float32),
            pltpu.VMEM((3, GC), jnp.float32),
            pltpu.VMEM((3, GC), jnp.float32),
            pltpu.VMEM((NP * PT,), jnp.float32),
            pltpu.SemaphoreType.DMA,
            pltpu.SemaphoreType.DMA,
            pltpu.SemaphoreType.DMA,
            pltpu.SemaphoreType.DMA,
            pltpu.SemaphoreType.DMA,
            pltpu.SemaphoreType.DMA,
            pltpu.SemaphoreType.DMA,
            pltpu.SemaphoreType.DMA,
            pltpu.SemaphoreType.DMA,
        ],
    )
    def sc_gather(a_hbm, b_hbm, p_hbm, dst_hbm, src_hbm,
                  ad_hbm, bs_hbm, rx_hbm, ry_hbm, rz_hbm,
                  idx_d, idx_s, buf_a, buf_b, buf_x, buf_y, buf_z,
                  ptab, gsem0, gsem1, gsem2, isem0, isem1, isem2,
                  wsem0, wsem1, wsem2):
        cid = jax.lax.axis_index("c")
        sid = jax.lax.axis_index("s")
        base = (cid * NS + sid) * EW
        pltpu.sync_copy(p_hbm, ptab)
        isem = (isem0, isem1, isem2)
        wsem = (wsem0, wsem1, wsem2)
        gsem = (gsem0, gsem1, gsem2)

        def fire_idx(ci, b):
            off = base + ci * GC
            pltpu.async_copy(dst_hbm.at[pl.ds(off, GC)], idx_d.at[b], isem[b])
            pltpu.async_copy(src_hbm.at[pl.ds(off, GC)], idx_s.at[b], isem[b])

        def drain_idx(ci, b):
            off = base + ci * GC
            pltpu.make_async_copy(dst_hbm.at[pl.ds(off, GC)], idx_d.at[b],
                                  isem[b]).wait()
            pltpu.make_async_copy(src_hbm.at[pl.ds(off, GC)], idx_s.at[b],
                                  isem[b]).wait()

        def fire_writes(ci, b):
            off = base + ci * GC
            pltpu.async_copy(buf_a.at[b], ad_hbm.at[pl.ds(off, GC)], wsem[b])
            pltpu.async_copy(buf_b.at[b], bs_hbm.at[pl.ds(off, GC)], wsem[b])
            pltpu.async_copy(buf_x.at[b], rx_hbm.at[pl.ds(off, GC)], wsem[b])
            pltpu.async_copy(buf_y.at[b], ry_hbm.at[pl.ds(off, GC)], wsem[b])
            pltpu.async_copy(buf_z.at[b], rz_hbm.at[pl.ds(off, GC)], wsem[b])

        def drain_writes(ci, b):
            off = base + ci * GC
            pltpu.make_async_copy(buf_a.at[b], ad_hbm.at[pl.ds(off, GC)],
                                  wsem[b]).wait()
            pltpu.make_async_copy(buf_b.at[b], bs_hbm.at[pl.ds(off, GC)],
                                  wsem[b]).wait()
            pltpu.make_async_copy(buf_x.at[b], rx_hbm.at[pl.ds(off, GC)],
                                  wsem[b]).wait()
            pltpu.make_async_copy(buf_y.at[b], ry_hbm.at[pl.ds(off, GC)],
                                  wsem[b]).wait()
            pltpu.make_async_copy(buf_z.at[b], rz_hbm.at[pl.ds(off, GC)],
                                  wsem[b]).wait()

        def fire_gathers(b):
            pltpu.async_copy(a_hbm.at[idx_d.at[b]], buf_a.at[b], gsem[b])
            pltpu.async_copy(b_hbm.at[idx_s.at[b]], buf_b.at[b], gsem[b])

        def drain_gathers(b):
            pltpu.make_async_copy(a_hbm.at[idx_d.at[b]], buf_a.at[b],
                                  gsem[b]).wait()
            pltpu.make_async_copy(b_hbm.at[idx_s.at[b]], buf_b.at[b],
                                  gsem[b]).wait()

        def body(ci, b):
            pb = (b + 2) % 3

            @pl.when(ci > 0)
            def _():
                drain_writes(ci - 1, pb)

            @pl.when(ci + 2 < NCH)
            def _():
                drain_idx(ci + 2, pb)
                fire_gathers(pb)

            drain_gathers(b)

            @pl.when(ci + 3 < NCH)
            def _():
                fire_idx(ci + 3, b)

            for k in range(GC // 16):
                sl = pl.ds(k * 16, 16)
                ird = idx_d[b, sl] * PT
                irs = idx_s[b, sl] * PT
                buf_x[b, sl] = (plsc.load_gather(ptab, [ird])
                                - plsc.load_gather(ptab, [irs]))
                buf_y[b, sl] = (plsc.load_gather(ptab, [ird + 1])
                                - plsc.load_gather(ptab, [irs + 1]))
                buf_z[b, sl] = (plsc.load_gather(ptab, [ird + 2])
                                - plsc.load_gather(ptab, [irs + 2]))

            fire_writes(ci, b)

        fire_idx(0, 0)
        fire_idx(1, 1)
        fire_idx(2, 2)
        drain_idx(0, 0)
        fire_gathers(0)
        drain_idx(1, 1)
        fire_gathers(1)

        @pl.loop(0, (NCH - 2) // 3)
        def _(k):
            body(3 * k, 0)
            body(3 * k + 1, 1)
            body(3 * k + 2, 2)

        body(NCH - 2, 0)
        body(NCH - 1, 1)
        drain_writes(NCH - 1, 1)

    mesh2 = plsc.VectorSubcoreMesh(core_axis_name="c", subcore_axis_name="s",
                                   num_cores=NC, num_subcores=KS)

    @functools.partial(
        pl.kernel,
        compiler_params=cp2,
        out_type=(
            jax.ShapeDtypeStruct((NC, NP, H), jnp.float32),
            jax.ShapeDtypeStruct((NC * KS, NP, H), jnp.float32),
        ),
        mesh=mesh2,
        scratch_types=[
            pltpu.VMEM((3, GS), jnp.int32),
            pltpu.VMEM((3, GS, SW), jnp.float32),
            pltpu.VMEM((3, GS, CW), jnp.float32),
            pltpu.VMEM((32, SW), jnp.float32),
            pltpu.VMEM_SHARED((NP, SW), jnp.float32),
            pltpu.VMEM_SHARED((NP, CW), jnp.float32),
            pltpu.SemaphoreType.DMA,
            pltpu.SemaphoreType.DMA,
            pltpu.SemaphoreType.DMA,
            pltpu.SemaphoreType.DMA,
            pltpu.SemaphoreType.DMA,
            pltpu.SemaphoreType.DMA,
            pltpu.SemaphoreType.DMA,
        ],
    )
    def sc_scatter(ef_hbm, cd_hbm, dst_hbm,
                   hpart_hbm, ppart_hbm,
                   idx_v, buf_ef, buf_cd, zbuf, acc_h, acc_p,
                   zsem, ssem0, ssem1, ssem2, lsem0, lsem1, lsem2):
        cid = jax.lax.axis_index("c")
        sid = jax.lax.axis_index("s")
        lsem = (lsem0, lsem1, lsem2)
        ssem = (ssem0, ssem1, ssem2)

        z16 = jnp.zeros((16,), jnp.float32)

        @pl.loop(0, 32)
        def _(j):
            @pl.loop(0, SW // 16)
            def _(q):
                zbuf[j, pl.ds(q * 16, 16)] = z16

        @pl.loop(0, NP // 32)
        def _(k):
            pltpu.async_copy(zbuf, acc_h.at[pl.ds(k * 32, 32)], zsem)
            pltpu.async_copy(zbuf.at[:, pl.ds(0, CW)],
                             acc_p.at[pl.ds(k * 32, 32)], zsem)

        @pl.loop(0, NP // 32)
        def _(k):
            pltpu.make_async_copy(zbuf, acc_h.at[pl.ds(k * 32, 32)],
                                  zsem).wait()
            pltpu.make_async_copy(zbuf.at[:, pl.ds(0, CW)],
                                  acc_p.at[pl.ds(k * 32, 32)], zsem).wait()

        ebase = cid * ECS
        lane0 = sid * SW

        def fire_loads(ci, b):
            off = ebase + ci * GS
            pltpu.async_copy(dst_hbm.at[pl.ds(off, GS)], idx_v.at[b], lsem[b])
            pltpu.async_copy(ef_hbm.at[pl.ds(off, GS), pl.ds(lane0, SW)],
                             buf_ef.at[b], lsem[b])
            pltpu.async_copy(cd_hbm.at[pl.ds(off, GS), pl.ds(0, CW)],
                             buf_cd.at[b], lsem[b])

        def drain_loads(ci, b):
            off = ebase + ci * GS
            pltpu.make_async_copy(dst_hbm.at[pl.ds(off, GS)], idx_v.at[b],
                                  lsem[b]).wait()
            pltpu.make_async_copy(ef_hbm.at[pl.ds(off, GS),
                                            pl.ds(lane0, SW)],
                                  buf_ef.at[b], lsem[b]).wait()
            pltpu.make_async_copy(cd_hbm.at[pl.ds(off, GS), pl.ds(0, CW)],
                                  buf_cd.at[b], lsem[b]).wait()

        def drain_streams(b):
            pltpu.make_async_copy(buf_ef.at[b], acc_h.at[idx_v.at[b]],
                                  ssem[b]).wait()
            pltpu.make_async_copy(buf_cd.at[b], acc_p.at[idx_v.at[b]],
                                  ssem[b]).wait()

        def body(ci, b):
            drain_loads(ci, b)
            pltpu.async_copy(buf_ef.at[b], acc_h.at[idx_v.at[b]],
                             ssem[b], add=True)
            pltpu.async_copy(buf_cd.at[b], acc_p.at[idx_v.at[b]],
                             ssem[b], add=True)
            pb = (b + 2) % 3

            @pl.when(ci > 0)
            def _():
                drain_streams(pb)

            @pl.when(ci + 2 < NCH2)
            def _():
                fire_loads(ci + 2, pb)

        fire_loads(0, 0)
        fire_loads(1, 1)

        @pl.loop(0, NCH2 // 3)
        def _(k):
            body(3 * k, 0)
            body(3 * k + 1, 1)
            body(3 * k + 2, 2)

        body(NCH2 - 1, 0)
        drain_streams(0)

        @pl.loop(0, NP // 640)
        def _(k):
            r = k * 640
            pltpu.sync_copy(acc_h.at[pl.ds(r, 640)],
                            hpart_hbm.at[cid, pl.ds(r, 640),
                                         pl.ds(lane0, SW)])
            pltpu.sync_copy(acc_p.at[pl.ds(r, 640)],
                            ppart_hbm.at[cid * KS + sid, pl.ds(r, 640),
                                         pl.ds(0, CW)])

    return sc_gather, sc_scatter


def _edge_body(ad, bs, rx, ry, rz, w1c, ew2, eb2, cw1, cb1, cw2r, ef_o, cd_o):
    vx = rx[0]
    vy = ry[0]
    vz = rz[0]
    dist = jnp.sqrt(vx * vx + vy * vy + vz * vz + 1e-8)
    dn = (((0,), (0,)), ((), ()))
    dterm = jax.lax.dot_general(dist, w1c[...], dn,
                                preferred_element_type=jnp.float32)
    x1 = _silu(ad[...] + bs[...] + dterm)
    ef = _silu(jnp.dot(x1, ew2[...], preferred_element_type=jnp.float32) + eb2[...])
    t = _silu(jnp.dot(ef, cw1[...], preferred_element_type=jnp.float32) + cb1[...])
    cw = jnp.sum(t * cw2r[...], axis=1, keepdims=True)
    rows = jnp.concatenate(
        [vx, vy, vz, jnp.zeros((5, vx.shape[1]), jnp.float32)], axis=0)
    ipad = jnp.eye(8, H, dtype=jnp.float32)
    rel_cols = jax.lax.dot_general(rows, ipad, dn,
                                   preferred_element_type=jnp.float32)
    ef_o[...] = ef
    cd_o[...] = cw * rel_cols


def _edge_call(ad, bs, rx3, ry3, rz3, w1c, ew2, eb2, cw1, cb1, cw2r):
    row = lambda i: (i, 0)
    full = lambda i: (0, 0)
    vec = lambda i: (i, 0, 0)
    return pl.pallas_call(
        _edge_body,
        grid=(NBE,),
        in_specs=[
            pl.BlockSpec((BE, H), row),
            pl.BlockSpec((BE, H), row),
            pl.BlockSpec((1, 1, BE), vec),
            pl.BlockSpec((1, 1, BE), vec),
            pl.BlockSpec((1, 1, BE), vec),
            pl.BlockSpec((1, H), full),
            pl.BlockSpec((H, H), full),
            pl.BlockSpec((1, H), full),
            pl.BlockSpec((H, H), full),
            pl.BlockSpec((1, H), full),
            pl.BlockSpec((1, H), full),
        ],
        out_specs=[
            pl.BlockSpec((BE, H), row),
            pl.BlockSpec((BE, H), row),
        ],
        out_shape=[
            jax.ShapeDtypeStruct((E, H), jnp.float32),
            jax.ShapeDtypeStruct((E, H), jnp.float32),
        ],
    )(ad, bs, rx3, ry3, rz3, w1c, ew2, eb2, cw1, cb1, cw2r)


def _precompute(hn, w1a, w1b, w1c, eb1, ew2, eb2):
    a = jnp.dot(hn, w1a[...], preferred_element_type=jnp.float32) + eb1[...]
    bm = jnp.dot(hn, w1b[...], preferred_element_type=jnp.float32)
    dist0 = jnp.sqrt(jnp.float32(1e-8))
    efl = _silu(jnp.dot(_silu(a + bm + dist0 * w1c[...]), ew2[...],
                        preferred_element_type=jnp.float32) + eb2[...])
    return a, bm, efl


def _embed_body(nf, inw, inb, w1a, w1b, w1c, eb1, ew2, eb2,
                h_o, a_o, b_o, efl_o):
    h = jnp.dot(nf[...], inw[...], preferred_element_type=jnp.float32) + inb[...]
    a, bm, efl = _precompute(h, w1a, w1b, w1c, eb1, ew2, eb2)
    h_o[...] = h
    a_o[...] = a
    b_o[...] = bm
    efl_o[...] = efl


def _embed_call(nf_p, inw, inb, w1a, w1b, w1c, eb1, ew2, eb2):
    row = lambda i: (i, 0)
    full = lambda i: (0, 0)
    return pl.pallas_call(
        _embed_body,
        grid=(NP // RN,),
        in_specs=[
            pl.BlockSpec((RN, H), row),
            pl.BlockSpec((H, H), full),
            pl.BlockSpec((1, H), full),
            pl.BlockSpec((H, H), full),
            pl.BlockSpec((H, H), full),
            pl.BlockSpec((1, H), full),
            pl.BlockSpec((1, H), full),
            pl.BlockSpec((H, H), full),
            pl.BlockSpec((1, H), full),
        ],
        out_specs=[pl.BlockSpec((RN, H), row)] * 4,
        out_shape=[jax.ShapeDtypeStruct((NP, H), jnp.float32)] * 4,
    )(nf_p, inw, inb, w1a, w1b, w1c, eb1, ew2, eb2)


def _node_update(h, hag, nw1a, nw1b, nb1, nw2, nb2, lng, lnb):
    u = _silu(jnp.dot(h, nw1a[...], preferred_element_type=jnp.float32)
              + jnp.dot(hag, nw1b[...], preferred_element_type=jnp.float32)
              + nb1[...])
    u = jnp.dot(u, nw2[...], preferred_element_type=jnp.float32) + nb2[...]
    x = h + h + u
    mu = jnp.mean(x, axis=1, keepdims=True)
    xc = x - mu
    var = jnp.mean(xc * xc, axis=1, keepdims=True)
    return xc * jax.lax.rsqrt(var + 1e-5) * lng[...] + lnb[...]


def _node_mid_body(h, hp0, hp1, efl, p, pp,
                   nw1a, nw1b, nb1, nw2, nb2, lng, lnb,
                   w1a, w1b, w1c, eb1, ew2, eb2,
                   h_o, p_o, a_o, b_o, efl_o):
    hag = hp0[0] + hp1[0] + efl[...]
    hn = _node_update(h[...], hag, nw1a, nw1b, nb1, nw2, nb2, lng, lnb)
    a, bm, efl2 = _precompute(hn, w1a, w1b, w1c, eb1, ew2, eb2)
    h_o[...] = hn
    p_o[...] = p[...] + jnp.sum(pp[...], axis=0)
    a_o[...] = a
    b_o[...] = bm
    efl_o[...] = efl2


def _node_mid_call(h, hparts, efl, p, pparts, nw1a, nw1b, nb1, nw2, nb2,
                   lng, lnb, w1a, w1b, w1c, eb1, ew2, eb2):
    row = lambda i: (i, 0)
    full = lambda i: (0, 0)
    part = lambda c: (lambda i: (c, i, 0))
    return pl.pallas_call(
        _node_mid_body,
        grid=(NP // RN,),
        in_specs=[
            pl.BlockSpec((RN, H), row),
            pl.BlockSpec((1, RN, H), part(0)),
            pl.BlockSpec((1, RN, H), part(1)),
            pl.BlockSpec((RN, H), row),
            pl.BlockSpec((RN, PT), row),
            pl.BlockSpec((NC * KS, RN, PT), lambda i: (0, i, 0)),
            pl.BlockSpec((H, H), full),
            pl.BlockSpec((H, H), full),
            pl.BlockSpec((1, H), full),
            pl.BlockSpec((H, H), full),
            pl.BlockSpec((1, H), full),
            pl.BlockSpec((1, H), full),
            pl.BlockSpec((1, H), full),
            pl.BlockSpec((H, H), full),
            pl.BlockSpec((H, H), full),
            pl.BlockSpec((1, H), full),
            pl.BlockSpec((1, H), full),
            pl.BlockSpec((H, H), full),
            pl.BlockSpec((1, H), full),
        ],
        out_specs=[
            pl.BlockSpec((RN, H), row),
            pl.BlockSpec((RN, PT), row),
            pl.BlockSpec((RN, H), row),
            pl.BlockSpec((RN, H), row),
            pl.BlockSpec((RN, H), row),
        ],
        out_shape=[
            jax.ShapeDtypeStruct((NP, H), jnp.float32),
            jax.ShapeDtypeStruct((NP, PT), jnp.float32),
            jax.ShapeDtypeStruct((NP, H), jnp.float32),
            jax.ShapeDtypeStruct((NP, H), jnp.float32),
            jax.ShapeDtypeStruct((NP, H), jnp.float32),
        ],
    )(h, hparts, hparts, efl, p, pparts,
      nw1a, nw1b, nb1, nw2, nb2, lng, lnb, w1a, w1b, w1c, eb1, ew2, eb2)


def _node_final_body(h, hp0, hp1, efl,
                     nw1a, nw1b, nb1, nw2, nb2, lng, lnb, h_o):
    hag = hp0[0] + hp1[0] + efl[...]
    h_o[...] = _node_update(h[...], hag, nw1a, nw1b, nb1, nw2, nb2, lng, lnb)


def _node_final_call(h, hparts, efl, nw1a, nw1b, nb1, nw2, nb2, lng, lnb):
    row = lambda i: (i, 0)
    full = lambda i: (0, 0)
    part = lambda c: (lambda i: (c, i, 0))
    return pl.pallas_call(
        _node_final_body,
        grid=(NP // RN,),
        in_specs=[
            pl.BlockSpec((RN, H), row),
            pl.BlockSpec((1, RN, H), part(0)),
            pl.BlockSpec((1, RN, H), part(1)),
            pl.BlockSpec((RN, H), row),
            pl.BlockSpec((H, H), full),
            pl.BlockSpec((H, H), full),
            pl.BlockSpec((1, H), full),
            pl.BlockSpec((H, H), full),
            pl.BlockSpec((1, H), full),
            pl.BlockSpec((1, H), full),
            pl.BlockSpec((1, H), full),
        ],
        out_specs=[pl.BlockSpec((RN, H), row)],
        out_shape=[jax.ShapeDtypeStruct((NP, H), jnp.float32)],
    )(h, hparts, hparts, efl, nw1a, nw1b, nb1, nw2, nb2, lng, lnb)[0]


def _pool_body(h, bat, outw, outb, o_ref, acc, cnt):
    i = pl.program_id(0)

    @pl.when(i == 0)
    def _():
        acc[...] = jnp.zeros_like(acc)
        cnt[...] = jnp.zeros_like(cnt)

    bb = bat[0]
    oh = (jax.lax.broadcasted_iota(jnp.int32, (B, RN), 0) == bb).astype(jnp.float32)
    acc[...] += jnp.dot(oh, h[...], preferred_element_type=jnp.float32)
    cnt[...] += jnp.sum(oh, axis=1, keepdims=True)

    @pl.when(i == pl.num_programs(0) - 1)
    def _():
        hg = acc[...] / cnt[...]
        o_ref[...] = jnp.dot(hg, outw[...],
                             preferred_element_type=jnp.float32) + outb[...]


def _pool_call(h, bat3, outw, outb):
    row = lambda i: (i, 0)
    full = lambda i: (0, 0)
    return pl.pallas_call(
        _pool_body,
        grid=(NP // RN,),
        in_specs=[
            pl.BlockSpec((RN, H), row),
            pl.BlockSpec((1, 1, RN), lambda i: (i, 0, 0)),
            pl.BlockSpec((H, H), full),
            pl.BlockSpec((1, H), full),
        ],
        out_specs=pl.BlockSpec((B, H), full),
        out_shape=jax.ShapeDtypeStruct((B, H), jnp.float32),
        scratch_shapes=[
            pltpu.VMEM((B, H), jnp.float32),
            pltpu.VMEM((B, 1), jnp.float32),
        ],
    )(h, bat3, outw, outb)


def kernel(node_features, pos, edge_index, batch, params):
    f32 = jnp.float32
    nf_p = jnp.pad(node_features, ((0, NP - N), (0, 0)))
    p4 = jnp.pad(pos.astype(f32), ((0, NP - N), (0, PT - 3)))
    bat3 = jnp.pad(batch.astype(jnp.int32), (0, NP - N),
                   constant_values=B).reshape(NP // RN, 1, RN)
    src = edge_index[0]
    dst = edge_index[1]

    def esplit(lp):
        w1 = lp['e_w1']
        return (w1[:H], w1[H:2 * H], w1[2 * H:2 * H + 1],
                lp['e_b1'].reshape(1, H), lp['e_w2'], lp['e_b2'].reshape(1, H))

    l0 = params['layers'][0]
    w1a, w1b, w1c, eb1, ew2, eb2 = esplit(l0)
    h, a, bm, efl = _embed_call(
        nf_p, params['in_w'], params['in_b'].reshape(1, H),
        w1a, w1b, w1c, eb1, ew2, eb2)
    p = p4

    for li in range(len(params['layers'])):
        lp = params['layers'][li]
        w1a, w1b, w1c, eb1, ew2, eb2 = esplit(lp)
        sc_gather, sc_scatter = _sc_kernels()
        ad, bs, rx, ry, rz = sc_gather(a, bm, p.reshape(NP * PT), dst, src)
        rx3 = rx.reshape(NBE, 1, BE)
        ry3 = ry.reshape(NBE, 1, BE)
        rz3 = rz.reshape(NBE, 1, BE)
        ef, cd = _edge_call(ad, bs, rx3, ry3, rz3, w1c, ew2, eb2,
                            lp['c_w1'], lp['c_b1'].reshape(1, H),
                            lp['c_w2'].reshape(1, H))
        hparts, pparts = sc_scatter(ef, cd, dst)
        pp4 = pparts[:, :, :PT]
        nw1 = lp['n_w1']
        args = (nw1[:H], nw1[H:2 * H], lp['n_b1'].reshape(1, H),
                lp['n_w2'], lp['n_b2'].reshape(1, H),
                lp['ln_g'].reshape(1, H), lp['ln_b'].reshape(1, H))
        if li + 1 < len(params['layers']):
            nxt = esplit(params['layers'][li + 1])
            h, p, a, bm, efl = _node_mid_call(h, hparts, efl, p, pp4,
                                             *args, *nxt)
        else:
            h = _node_final_call(h, hparts, efl, *args)

    return _pool_call(h, bat3, params['out_w'],
                      params['out_b'].reshape(1, H))
